# baseline (speedup 1.0000x reference)
import functools

import jax
import jax.numpy as jnp
from jax import lax
from jax.experimental import pallas as pl
from jax.experimental.pallas import tpu as pltpu
from jax.experimental.pallas import tpu_sc as plsc

B = 16384
F = 64
NC = 2
NS = 16
NW = NC * NS
BPW = B // NW

@functools.lru_cache(maxsize=1)
def _make_gather3():
    mesh = plsc.VectorSubcoreMesh(core_axis_name="c", subcore_axis_name="s")

    @functools.partial(
        pl.kernel,
        mesh=mesh,
        compiler_params=pltpu.CompilerParams(use_tc_tiling_on_sc=False),
        out_type=[
            jax.ShapeDtypeStruct((B, F), jnp.float32),
            jax.ShapeDtypeStruct((B, F), jnp.float32),
            jax.ShapeDtypeStruct((B, F), jnp.float32),
        ],
        scratch_types=[
            pltpu.VMEM((BPW,), jnp.int32),
            pltpu.VMEM((BPW,), jnp.int32),
            pltpu.VMEM((BPW,), jnp.int32),
            pltpu.VMEM((BPW, F), jnp.float32),
            pltpu.VMEM((BPW, F), jnp.float32),
            pltpu.VMEM((BPW, F), jnp.float32),
            pltpu.SemaphoreType.DMA,
        ],
    )
    def _gather3(user_t, movie_t, genre_t, uidx, midx, gidx,
                 out_u, out_m, out_g, uiv, miv, giv, urv, mrv, grv, sem):
        wid = lax.axis_index("s") * NC + lax.axis_index("c")
        base = wid * BPW
        pltpu.sync_copy(uidx.at[pl.ds(base, BPW)], uiv)
        pltpu.sync_copy(midx.at[pl.ds(base, BPW)], miv)
        pltpu.sync_copy(gidx.at[pl.ds(base, BPW)], giv)
        cu = pltpu.async_copy(user_t.at[uiv], urv, sem)
        cm = pltpu.async_copy(movie_t.at[miv], mrv, sem)
        cg = pltpu.async_copy(genre_t.at[giv], grv, sem)
        cu.wait()
        cm.wait()
        cg.wait()
        pltpu.sync_copy(urv, out_u.at[pl.ds(base, BPW)])
        pltpu.sync_copy(mrv, out_m.at[pl.ds(base, BPW)])
        pltpu.sync_copy(grv, out_g.at[pl.ds(base, BPW)])

    return _gather3


BT = 2048


def _mlp_body(ue, me, ge, w1u, w1m, w1g, b1, w2, b2, w3, b3, out):
    x = (jnp.dot(ue[...], w1u[...], preferred_element_type=jnp.float32)
         + jnp.dot(me[...], w1m[...], preferred_element_type=jnp.float32)
         + jnp.dot(ge[...], w1g[...], preferred_element_type=jnp.float32)
         + b1[...])
    x = jnp.maximum(x, 0.0)
    x = jnp.maximum(
        jnp.dot(x, w2[...], preferred_element_type=jnp.float32) + b2[...], 0.0)
    out[...] = jnp.dot(x, w3[...], preferred_element_type=jnp.float32) + b3[...]


def _mlp(ue, me, ge, w1u, w1m, w1g, b1, w2, b2, w3, b3, *, interpret=False):
    grid = B // BT
    full = lambda shape: pl.BlockSpec(shape, lambda i: (0, 0))
    return pl.pallas_call(
        _mlp_body,
        grid=(grid,),
        in_specs=[
            pl.BlockSpec((BT, F), lambda i: (i, 0)),
            pl.BlockSpec((BT, F), lambda i: (i, 0)),
            pl.BlockSpec((BT, F), lambda i: (i, 0)),
            full((F, 128)),
            full((F, 128)),
            full((F, 128)),
            full((1, 128)),
            full((128, F)),
            full((1, F)),
            full((F, 1)),
            full((1, 1)),
        ],
        out_specs=pl.BlockSpec((BT, 1), lambda i: (i, 0)),
        out_shape=jax.ShapeDtypeStruct((B, 1), jnp.float32),
        interpret=interpret,
    )(ue, me, ge, w1u, w1m, w1g, b1, w2, b2, w3, b3)


def kernel(user, movie, genres, user_table, movie_table, genre_table,
           W1, b1, W2, b2, W3, b3):
    ue, me, ge = _make_gather3()(user_table, movie_table, genre_table,
                                 user, movie, genres)
    w1u = W1[:, :F].T
    w1m = W1[:, F:2 * F].T
    w1g = W1[:, 2 * F:].T
    return _mlp(ue, me, ge, w1u, w1m, w1g,
                b1.reshape(1, 128), W2.T, b2.reshape(1, F),
                W3.T, b3.reshape(1, 1))
